# Initial kernel scaffold; baseline (speedup 1.0000x reference)
import jax
import jax.numpy as jnp
from jax.experimental import pallas as pl


def kernel(x, edge_index, edge_weight, batch, W1a, b1a, W1b, b1b, W2a, b2a, W2b, b2b, Wfc, bfc):
    raise NotImplementedError("write your pallas kernel here")


# speedup vs baseline: 3.5365x; 3.5365x over previous
import functools

import jax
import jax.numpy as jnp
from jax import lax
from jax.experimental import pallas as pl
from jax.experimental.pallas import tpu as pltpu
from jax.experimental.pallas import tpu_sc as plsc

N = 10000
E = 320000
D = 128
G = 64

NPAD = 10112
NC = 2
NS = 16
NW = NC * NS
EC = 2560
CPW = EC // NW
K = 2
ROWS_PER_TILE = NPAD // NS

BLK = 2528
NBLK = NPAD // BLK


def _sc_scatter_body(x_hbm, src_hbm, dst_hbm, zeros_hbm, out_hbm,
                     srcv, dstv, rows, agg_sh, sem):
    cid = lax.axis_index("c")
    sid = lax.axis_index("s")
    wid = sid * NC + cid

    pltpu.sync_copy(zeros_hbm, agg_sh.at[pl.ds(sid * ROWS_PER_TILE, ROWS_PER_TILE)])
    plsc.subcore_barrier()

    row0 = wid * CPW

    def step(g, _):
        base = row0 + g * K
        pltpu.sync_copy(src_hbm.at[pl.ds(base, K)], srcv)
        pltpu.sync_copy(dst_hbm.at[pl.ds(base, K)], dstv)
        cps = [pltpu.async_copy(x_hbm.at[srcv.at[j]], rows.at[j], sem)
               for j in range(K)]
        for c in cps:
            c.wait()
        for j in range(K):
            pltpu.sync_copy(rows.at[j], agg_sh.at[dstv.at[j]], add=True)
        return 0

    lax.fori_loop(0, CPW // K, step, 0)

    plsc.subcore_barrier()
    pltpu.sync_copy(agg_sh.at[pl.ds(sid * ROWS_PER_TILE, ROWS_PER_TILE)],
                    out_hbm.at[cid, pl.ds(sid * ROWS_PER_TILE, ROWS_PER_TILE)])


_sc_scatter = functools.partial(
    pl.kernel,
    out_type=jax.ShapeDtypeStruct((NC, NPAD, D), jnp.float32),
    mesh=plsc.VectorSubcoreMesh(core_axis_name="c", subcore_axis_name="s"),
    scratch_types=[
        pltpu.VMEM((K, 128), jnp.int32),
        pltpu.VMEM((K, 128), jnp.int32),
        pltpu.VMEM((K, 128, D), jnp.float32),
        pltpu.VMEM_SHARED((NPAD, D), jnp.float32),
        pltpu.SemaphoreType.DMA,
    ],
)(_sc_scatter_body)


def _mlp_body(x_ref, a0_ref, a1_ref, wa_ref, ba_ref, wb_ref, bb_ref, o_ref):
    z = x_ref[...] + a0_ref[...] + a1_ref[...]
    t = jnp.maximum(
        jnp.dot(z, wa_ref[...], preferred_element_type=jnp.float32) + ba_ref[...], 0.0)
    o_ref[...] = jnp.maximum(
        jnp.dot(t, wb_ref[...], preferred_element_type=jnp.float32) + bb_ref[...], 0.0)


def _mlp1(x, a0, a1, wa, ba, wb, bb):
    full = lambda i: (0, 0)
    return pl.pallas_call(
        _mlp_body,
        grid=(NBLK,),
        in_specs=[
            pl.BlockSpec((BLK, D), lambda i: (i, 0)),
            pl.BlockSpec((BLK, D), lambda i: (i, 0)),
            pl.BlockSpec((BLK, D), lambda i: (i, 0)),
            pl.BlockSpec((D, D), full),
            pl.BlockSpec((1, D), full),
            pl.BlockSpec((D, D), full),
            pl.BlockSpec((1, D), full),
        ],
        out_specs=pl.BlockSpec((BLK, D), lambda i: (i, 0)),
        out_shape=jax.ShapeDtypeStruct((NPAD, D), jnp.float32),
    )(x, a0, a1, wa, ba, wb, bb)


def _mlp2_pool_body(h_ref, a0_ref, a1_ref, wa_ref, ba_ref, wb_ref, bb_ref,
                    bat_ref, wfc_ref, bfc_ref, o_ref, acc, cnt):
    i = pl.program_id(0)

    @pl.when(i == 0)
    def _():
        acc[...] = jnp.zeros_like(acc)
        cnt[...] = jnp.zeros_like(cnt)

    z = h_ref[...] + a0_ref[...] + a1_ref[...]
    t = jnp.maximum(
        jnp.dot(z, wa_ref[...], preferred_element_type=jnp.float32) + ba_ref[...], 0.0)
    h2 = jnp.maximum(
        jnp.dot(t, wb_ref[...], preferred_element_type=jnp.float32) + bb_ref[...], 0.0)

    oh = (bat_ref[...] == lax.broadcasted_iota(jnp.int32, (BLK, 128), 1)
          ).astype(jnp.float32)
    dnum = (((0,), (0,)), ((), ()))
    acc[...] += lax.dot_general(oh, h2, dnum, preferred_element_type=jnp.float32)
    cnt[...] += lax.dot_general(oh, jnp.ones((BLK, 128), jnp.float32), dnum,
                                preferred_element_type=jnp.float32)

    @pl.when(i == NBLK - 1)
    def _():
        pooled = acc[...] / jnp.maximum(cnt[...], 1.0)
        o_ref[...] = (jnp.dot(pooled, wfc_ref[...],
                              preferred_element_type=jnp.float32) + bfc_ref[...])


def _mlp2_pool(h, a0, a1, wa, ba, wb, bb, bat, wfc, bfc):
    full = lambda i: (0, 0)
    return pl.pallas_call(
        _mlp2_pool_body,
        grid=(NBLK,),
        in_specs=[
            pl.BlockSpec((BLK, D), lambda i: (i, 0)),
            pl.BlockSpec((BLK, D), lambda i: (i, 0)),
            pl.BlockSpec((BLK, D), lambda i: (i, 0)),
            pl.BlockSpec((D, D), full),
            pl.BlockSpec((1, D), full),
            pl.BlockSpec((D, D), full),
            pl.BlockSpec((1, D), full),
            pl.BlockSpec((BLK, 1), lambda i: (i, 0)),
            pl.BlockSpec((D, 1), full),
            pl.BlockSpec((1, 1), full),
        ],
        out_specs=pl.BlockSpec((128, 1), full),
        out_shape=jax.ShapeDtypeStruct((128, 1), jnp.float32),
        scratch_shapes=[
            pltpu.VMEM((128, 128), jnp.float32),
            pltpu.VMEM((128, 128), jnp.float32),
        ],
    )(h, a0, a1, wa, ba, wb, bb, bat, wfc, bfc)


def kernel(x, edge_index, edge_weight, batch,
           W1a, b1a, W1b, b1b, W2a, b2a, W2b, b2b, Wfc, bfc):
    del edge_weight

    x_pad = jnp.zeros((NPAD, D), jnp.float32).at[:N].set(x)
    epad = EC * 128 - E
    src = jnp.concatenate([edge_index[0], jnp.zeros((epad,), jnp.int32)]).reshape(EC, 128)
    dst = jnp.concatenate([edge_index[1], jnp.full((epad,), N, jnp.int32)]).reshape(EC, 128)
    zeros_tile = jnp.zeros((ROWS_PER_TILE, D), jnp.float32)
    bat = jnp.concatenate([batch, jnp.full((NPAD - N,), G, jnp.int32)]).reshape(NPAD, 1)
    b1a2, b1b2 = b1a.reshape(1, D), b1b.reshape(1, D)
    b2a2, b2b2 = b2a.reshape(1, D), b2b.reshape(1, D)
    bfc2 = bfc.reshape(1, 1)

    agg = _sc_scatter(x_pad, src, dst, zeros_tile)
    h = _mlp1(x_pad, agg[0], agg[1], W1a, b1a2, W1b, b1b2)

    agg2 = _sc_scatter(h, src, dst, zeros_tile)
    pred = _mlp2_pool(h, agg2[0], agg2[1], W2a, b2a2, W2b, b2b2, bat, Wfc, bfc2)

    return pred[:G]


# speedup vs baseline: 4.1378x; 1.1700x over previous
import functools

import jax
import jax.numpy as jnp
from jax import lax
from jax.experimental import pallas as pl
from jax.experimental.pallas import tpu as pltpu
from jax.experimental.pallas import tpu_sc as plsc

N = 10000
E = 320000
D = 128
G = 64

NPAD = 10112
NC = 2
NS = 16
NW = NC * NS
EC = 2560
CPW = EC // NW
B = 8
ROWS_PER_TILE = NPAD // NS

BLK = 2528
NBLK = NPAD // BLK


def _sc_scatter_body(x_hbm, src_hbm, dst_hbm, zeros_hbm, out_hbm,
                     srcv, dstv, rows, agg_sh, sem):
    cid = lax.axis_index("c")
    sid = lax.axis_index("s")
    wid = sid * NC + cid

    pltpu.sync_copy(zeros_hbm, agg_sh.at[pl.ds(sid * ROWS_PER_TILE, ROWS_PER_TILE)])
    plsc.subcore_barrier()

    row0 = wid * CPW

    def _drain(slot):
        pltpu.make_async_copy(x_hbm.at[pl.ds(0, 128)], rows.at[slot], sem).wait()

    def block(t, _):
        @pl.when(t > 0)
        def _():
            _drain((B - 1) % 2)
            pltpu.sync_copy(rows.at[(B - 1) % 2], agg_sh.at[dstv.at[B - 1]],
                            add=True)

        base = row0 + t * B
        pltpu.sync_copy(src_hbm.at[pl.ds(base, B)], srcv)
        pltpu.sync_copy(dst_hbm.at[pl.ds(base, B)], dstv)
        for j in range(B):
            pltpu.async_copy(x_hbm.at[srcv.at[j]], rows.at[j % 2], sem)
            if j > 0:
                _drain((j - 1) % 2)
                pltpu.sync_copy(rows.at[(j - 1) % 2],
                                agg_sh.at[dstv.at[j - 1]], add=True)
        return 0

    lax.fori_loop(0, CPW // B, block, 0)
    _drain((B - 1) % 2)
    pltpu.sync_copy(rows.at[(B - 1) % 2], agg_sh.at[dstv.at[B - 1]], add=True)

    plsc.subcore_barrier()
    pltpu.sync_copy(agg_sh.at[pl.ds(sid * ROWS_PER_TILE, ROWS_PER_TILE)],
                    out_hbm.at[cid, pl.ds(sid * ROWS_PER_TILE, ROWS_PER_TILE)])


_sc_scatter = functools.partial(
    pl.kernel,
    out_type=jax.ShapeDtypeStruct((NC, NPAD, D), jnp.float32),
    mesh=plsc.VectorSubcoreMesh(core_axis_name="c", subcore_axis_name="s"),
    scratch_types=[
        pltpu.VMEM((B, 128), jnp.int32),
        pltpu.VMEM((B, 128), jnp.int32),
        pltpu.VMEM((2, 128, D), jnp.float32),
        pltpu.VMEM_SHARED((NPAD, D), jnp.float32),
        pltpu.SemaphoreType.DMA,
    ],
)(_sc_scatter_body)


def _mlp_body(x_ref, a0_ref, a1_ref, wa_ref, ba_ref, wb_ref, bb_ref, o_ref):
    z = x_ref[...] + a0_ref[...] + a1_ref[...]
    t = jnp.maximum(
        jnp.dot(z, wa_ref[...], preferred_element_type=jnp.float32) + ba_ref[...], 0.0)
    o_ref[...] = jnp.maximum(
        jnp.dot(t, wb_ref[...], preferred_element_type=jnp.float32) + bb_ref[...], 0.0)


def _mlp1(x, a0, a1, wa, ba, wb, bb):
    full = lambda i: (0, 0)
    return pl.pallas_call(
        _mlp_body,
        grid=(NBLK,),
        in_specs=[
            pl.BlockSpec((BLK, D), lambda i: (i, 0)),
            pl.BlockSpec((BLK, D), lambda i: (i, 0)),
            pl.BlockSpec((BLK, D), lambda i: (i, 0)),
            pl.BlockSpec((D, D), full),
            pl.BlockSpec((1, D), full),
            pl.BlockSpec((D, D), full),
            pl.BlockSpec((1, D), full),
        ],
        out_specs=pl.BlockSpec((BLK, D), lambda i: (i, 0)),
        out_shape=jax.ShapeDtypeStruct((NPAD, D), jnp.float32),
    )(x, a0, a1, wa, ba, wb, bb)


def _mlp2_pool_body(h_ref, a0_ref, a1_ref, wa_ref, ba_ref, wb_ref, bb_ref,
                    bat_ref, wfc_ref, bfc_ref, o_ref, acc, cnt):
    i = pl.program_id(0)

    @pl.when(i == 0)
    def _():
        acc[...] = jnp.zeros_like(acc)
        cnt[...] = jnp.zeros_like(cnt)

    z = h_ref[...] + a0_ref[...] + a1_ref[...]
    t = jnp.maximum(
        jnp.dot(z, wa_ref[...], preferred_element_type=jnp.float32) + ba_ref[...], 0.0)
    h2 = jnp.maximum(
        jnp.dot(t, wb_ref[...], preferred_element_type=jnp.float32) + bb_ref[...], 0.0)

    oh = (bat_ref[...] == lax.broadcasted_iota(jnp.int32, (BLK, 128), 1)
          ).astype(jnp.float32)
    dnum = (((0,), (0,)), ((), ()))
    acc[...] += lax.dot_general(oh, h2, dnum, preferred_element_type=jnp.float32)
    cnt[...] += lax.dot_general(oh, jnp.ones((BLK, 128), jnp.float32), dnum,
                                preferred_element_type=jnp.float32)

    @pl.when(i == NBLK - 1)
    def _():
        pooled = acc[...] / jnp.maximum(cnt[...], 1.0)
        o_ref[...] = (jnp.dot(pooled, wfc_ref[...],
                              preferred_element_type=jnp.float32) + bfc_ref[...])


def _mlp2_pool(h, a0, a1, wa, ba, wb, bb, bat, wfc, bfc):
    full = lambda i: (0, 0)
    return pl.pallas_call(
        _mlp2_pool_body,
        grid=(NBLK,),
        in_specs=[
            pl.BlockSpec((BLK, D), lambda i: (i, 0)),
            pl.BlockSpec((BLK, D), lambda i: (i, 0)),
            pl.BlockSpec((BLK, D), lambda i: (i, 0)),
            pl.BlockSpec((D, D), full),
            pl.BlockSpec((1, D), full),
            pl.BlockSpec((D, D), full),
            pl.BlockSpec((1, D), full),
            pl.BlockSpec((BLK, 1), lambda i: (i, 0)),
            pl.BlockSpec((D, 1), full),
            pl.BlockSpec((1, 1), full),
        ],
        out_specs=pl.BlockSpec((128, 1), full),
        out_shape=jax.ShapeDtypeStruct((128, 1), jnp.float32),
        scratch_shapes=[
            pltpu.VMEM((128, 128), jnp.float32),
            pltpu.VMEM((128, 128), jnp.float32),
        ],
    )(h, a0, a1, wa, ba, wb, bb, bat, wfc, bfc)


def kernel(x, edge_index, edge_weight, batch,
           W1a, b1a, W1b, b1b, W2a, b2a, W2b, b2b, Wfc, bfc):
    del edge_weight

    x_pad = jnp.zeros((NPAD, D), jnp.float32).at[:N].set(x)
    epad = EC * 128 - E
    src = jnp.concatenate([edge_index[0], jnp.zeros((epad,), jnp.int32)]).reshape(EC, 128)
    dst = jnp.concatenate([edge_index[1], jnp.full((epad,), N, jnp.int32)]).reshape(EC, 128)
    zeros_tile = jnp.zeros((ROWS_PER_TILE, D), jnp.float32)
    bat = jnp.concatenate([batch, jnp.full((NPAD - N,), G, jnp.int32)]).reshape(NPAD, 1)
    b1a2, b1b2 = b1a.reshape(1, D), b1b.reshape(1, D)
    b2a2, b2b2 = b2a.reshape(1, D), b2b.reshape(1, D)
    bfc2 = bfc.reshape(1, 1)

    agg = _sc_scatter(x_pad, src, dst, zeros_tile)
    h = _mlp1(x_pad, agg[0], agg[1], W1a, b1a2, W1b, b1b2)

    agg2 = _sc_scatter(h, src, dst, zeros_tile)
    pred = _mlp2_pool(h, agg2[0], agg2[1], W2a, b2a2, W2b, b2b2, bat, Wfc, bfc2)

    return pred[:G]
